# baseline (speedup 1.0000x reference)
import jax
import jax.numpy as jnp
from jax.experimental import pallas as pl

EMB = 4096
NE = 64
K = 8
NT = 8192
M_BLK = 1024
SUB = 256


def _router_block(x_ref, w_ref, probs_ref, idx_ref, scores_ref):
    w = w_ref[...]
    colsf = (jnp.int32(NE - 1) - jax.lax.broadcasted_iota(jnp.int32, (SUB, NE), 1)
             ).astype(jnp.float32)
    neg_one = jnp.float32(-1.0)
    neg_inf = jnp.float32(-jnp.inf)
    for c in range(M_BLK // SUB):
        x = x_ref[c * SUB:(c + 1) * SUB, :]
        work = jax.lax.dot_general(
            x, w, (((1,), (1,)), ((), ())), preferred_element_type=jnp.float32
        )
        scores_ref[c * SUB:(c + 1) * SUB, :] = work
        vals = []
        revs = []
        for r in range(K):
            m = jnp.max(work, axis=1, keepdims=True)
            cand = jnp.where(work == m, colsf, neg_one)
            jm = jnp.max(cand, axis=1, keepdims=True)
            vals.append(m)
            revs.append(jm)
            if r < K - 1:
                work = jnp.where(cand == jm, neg_inf, work)
        top = jnp.concatenate(vals, axis=1)
        rev = jnp.concatenate(revs, axis=1)
        top_idx = jnp.int32(NE - 1) - rev.astype(jnp.int32)
        e = jnp.exp(top - top[:, 0:1])
        probs = e / jnp.sum(e, axis=1, keepdims=True)
        probs_ref[c * SUB:(c + 1) * SUB, :] = probs
        idx_ref[c * SUB:(c + 1) * SUB, :] = top_idx


@jax.jit
def kernel(x, W_gate):
    grid = (NT // M_BLK,)
    probs, idx, scores = pl.pallas_call(
        _router_block,
        grid=grid,
        in_specs=[
            pl.BlockSpec((M_BLK, EMB), lambda i: (i, 0)),
            pl.BlockSpec((NE, EMB), lambda i: (0, 0)),
        ],
        out_specs=[
            pl.BlockSpec((M_BLK, K), lambda i: (i, 0)),
            pl.BlockSpec((M_BLK, K), lambda i: (i, 0)),
            pl.BlockSpec((M_BLK, NE), lambda i: (i, 0)),
        ],
        out_shape=[
            jax.ShapeDtypeStruct((NT, K), jnp.float32),
            jax.ShapeDtypeStruct((NT, K), jnp.int32),
            jax.ShapeDtypeStruct((NT, NE), jnp.float32),
        ],
    )(x, W_gate)
    return (probs, idx, scores)
